# baseline (speedup 1.0000x reference)
import functools

import jax
import jax.numpy as jnp
from jax.experimental import pallas as pl
from jax.experimental.pallas import tpu as pltpu

N_W = 10000
N_P = 10000
E = 320000
IN_DIM = 128
OUT_DIM = 128
H = 8
DH = OUT_DIM // H
FEAT = 16
FFN = 512

BM = 1000
BE = 12800


def _zs_body(w_ref, fcw_ref, a1_ref, zs_ref, smax_ref):
    z = jnp.dot(w_ref[...], fcw_ref[...], preferred_element_type=jnp.float32)
    s = jnp.dot(z, a1_ref[...], preferred_element_type=jnp.float32)
    zs_ref[:, :IN_DIM] = z
    zs_ref[:, IN_DIM:] = s
    blk = jnp.broadcast_to(jnp.max(s, axis=0, keepdims=True), (8, 16))

    @pl.when(pl.program_id(0) == 0)
    def _():
        smax_ref[...] = blk

    @pl.when(pl.program_id(0) != 0)
    def _():
        smax_ref[...] = jnp.maximum(smax_ref[...], blk)


def _cfeat_body(ef_ref, wc_ref, bc_ref, ct_ref, cmax_ref):
    ct = jax.lax.dot_general(
        wc_ref[...], ef_ref[...], (((0,), (1,)), ((), ())),
        preferred_element_type=jnp.float32)
    ct = ct + bc_ref[...].reshape(16, 1)
    ct_ref[...] = ct
    blk = jnp.broadcast_to(jnp.max(ct, axis=1, keepdims=True), (16, 128))

    @pl.when(pl.program_id(0) == 0)
    def _():
        cmax_ref[...] = blk

    @pl.when(pl.program_id(0) != 0)
    def _():
        cmax_ref[...] = jnp.maximum(cmax_ref[...], blk)


def _ffn_body(num_ref, den_ref, p_ref, exp_ref, w1_ref, b1_ref, w2_ref,
              b2_ref, g_ref, b_ref, out_ref):
    den_exp = jnp.dot(den_ref[...], exp_ref[...],
                      preferred_element_type=jnp.float32)
    g = num_ref[...] / (den_exp + 1e-20)
    g = jnp.where(g > 0, g, jnp.exp(jnp.minimum(g, 0.0)) - 1.0)
    h = g + p_ref[...]
    mu = jnp.mean(h, axis=-1, keepdims=True)
    var = jnp.mean(jnp.square(h - mu), axis=-1, keepdims=True)
    xln = (h - mu) / jnp.sqrt(var + 1e-5) * g_ref[...] + b_ref[...]
    inter = jnp.dot(xln, w1_ref[...], preferred_element_type=jnp.float32)
    inter = jnp.maximum(inter + b1_ref[...], 0.0)
    out = jnp.dot(inter, w2_ref[...], preferred_element_type=jnp.float32)
    out_ref[...] = out + b2_ref[...] + h


def _dense_pre(w, edge_feat, fcW, attnW, featW, featB):
    a1 = attnW[:, :DH]
    a2 = attnW[:, DH:]
    fcw_cat = jnp.transpose(fcW, (1, 0, 2)).reshape(IN_DIM, H * DH)
    a1blk = jnp.zeros((H * DH, 16), jnp.float32)
    a1blk = a1blk.at[jnp.arange(H * DH), jnp.arange(H * DH) // DH].set(
        a1.reshape(-1))
    wc = jnp.einsum("hfd,hd->fh", featW, a2)
    wc16 = jnp.concatenate([wc, jnp.zeros((FEAT, 8), jnp.float32)], axis=1)
    bc = jnp.sum(featB * a2, axis=1)
    bc16 = jnp.concatenate([bc, jnp.zeros((8,), jnp.float32)])

    nb = N_W // BM
    zs, smax = pl.pallas_call(
        _zs_body,
        grid=(nb,),
        in_specs=[
            pl.BlockSpec((BM, IN_DIM), lambda i: (i, 0)),
            pl.BlockSpec((IN_DIM, H * DH), lambda i: (0, 0)),
            pl.BlockSpec((H * DH, 16), lambda i: (0, 0)),
        ],
        out_specs=[
            pl.BlockSpec((BM, IN_DIM + 16), lambda i: (i, 0)),
            pl.BlockSpec((8, 16), lambda i: (0, 0)),
        ],
        out_shape=[
            jax.ShapeDtypeStruct((N_W, IN_DIM + 16), jnp.float32),
            jax.ShapeDtypeStruct((8, 16), jnp.float32),
        ],
    )(w, fcw_cat, a1blk)

    ne = E // BE
    ct, cmax = pl.pallas_call(
        _cfeat_body,
        grid=(ne,),
        in_specs=[
            pl.BlockSpec((BE, FEAT), lambda i: (i, 0)),
            pl.BlockSpec((FEAT, 16), lambda i: (0, 0)),
            pl.BlockSpec((1, 16), lambda i: (0, 0)),
        ],
        out_specs=[
            pl.BlockSpec((16, BE), lambda i: (0, i)),
            pl.BlockSpec((16, 128), lambda i: (0, 0)),
        ],
        out_shape=[
            jax.ShapeDtypeStruct((16, E), jnp.float32),
            jax.ShapeDtypeStruct((16, 128), jnp.float32),
        ],
    )(edge_feat, wc16, bc16.reshape(1, 16))

    s_max = smax[0, :H]
    c_max = cmax[:H, 0]
    m = s_max + c_max
    m = jnp.where(m >= 0, m, 0.01 * m)
    return zs, ct, m


def _dense_post(num, den16, p, w1W, w1b, w2W, w2b, ln_g, ln_b):
    expand = jnp.zeros((16, OUT_DIM), jnp.float32)
    expand = expand.at[jnp.arange(OUT_DIM) // DH, jnp.arange(OUT_DIM)].set(1.0)
    nb = N_P // BM
    return pl.pallas_call(
        _ffn_body,
        grid=(nb,),
        in_specs=[
            pl.BlockSpec((BM, OUT_DIM), lambda i: (i, 0)),
            pl.BlockSpec((BM, 16), lambda i: (i, 0)),
            pl.BlockSpec((BM, IN_DIM), lambda i: (i, 0)),
            pl.BlockSpec((16, OUT_DIM), lambda i: (0, 0)),
            pl.BlockSpec((OUT_DIM, FFN), lambda i: (0, 0)),
            pl.BlockSpec((1, FFN), lambda i: (0, 0)),
            pl.BlockSpec((FFN, OUT_DIM), lambda i: (0, 0)),
            pl.BlockSpec((1, OUT_DIM), lambda i: (0, 0)),
            pl.BlockSpec((1, OUT_DIM), lambda i: (0, 0)),
            pl.BlockSpec((1, OUT_DIM), lambda i: (0, 0)),
        ],
        out_specs=pl.BlockSpec((BM, OUT_DIM), lambda i: (i, 0)),
        out_shape=jax.ShapeDtypeStruct((N_P, OUT_DIM), jnp.float32),
    )(num, den16, p, expand, w1W, w1b.reshape(1, FFN), w2W,
      w2b.reshape(1, OUT_DIM), ln_g.reshape(1, OUT_DIM),
      ln_b.reshape(1, OUT_DIM))


def _edge_pass(zs, ct, m, src, dst):
    s_src = zs[src, IN_DIM:IN_DIM + H]
    e = s_src + ct[:H, :].T
    e = jnp.where(e >= 0, e, 0.01 * e)
    ex = jnp.exp(e - m[None, :])
    den = jax.ops.segment_sum(ex, dst, num_segments=N_P)
    z_src = zs[src, :IN_DIM].reshape(E, H, DH)
    num = jax.ops.segment_sum(
        (ex[:, :, None] * z_src).reshape(E, OUT_DIM), dst, num_segments=N_P)
    den16 = jnp.concatenate([den, jnp.zeros((N_P, 8), jnp.float32)], axis=1)
    return num, den16


def kernel(w, p, edge_feat, fcW, attnW, featW, featB, w1W, w1b, w2W, w2b,
           ln_g, ln_b, edge_index):
    src = edge_index[0].astype(jnp.int32)
    dst = edge_index[1].astype(jnp.int32)
    zs, ct, m = _dense_pre(w, edge_feat, fcW, attnW, featW, featB)
    num, den16 = _edge_pass(zs, ct, m, src, dst)
    return _dense_post(num, den16, p, w1W, w1b, w2W, w2b, ln_g, ln_b)


# speedup vs baseline: 157.8470x; 157.8470x over previous
import functools

import jax
import jax.numpy as jnp
from jax import lax
from jax.experimental import pallas as pl
from jax.experimental.pallas import tpu as pltpu
from jax.experimental.pallas import tpu_sc as plsc

N_W = 10000
N_P = 10000
E = 320000
IN_DIM = 128
OUT_DIM = 128
H = 8
DH = OUT_DIM // H
FEAT = 16
FFN = 512

BM = 1000
BE = 12800


def _zs_body(w_ref, fcw_ref, a1_ref, zs_ref, smax_ref):
    z = jnp.dot(w_ref[...], fcw_ref[...], preferred_element_type=jnp.float32)
    s = jnp.dot(z, a1_ref[...], preferred_element_type=jnp.float32)
    zs_ref[:, :IN_DIM] = z
    zs_ref[:, IN_DIM:IN_DIM + 16] = s
    zs_ref[:, IN_DIM + 16:] = jnp.zeros((BM, 112), jnp.float32)
    blk = jnp.broadcast_to(jnp.max(s, axis=0, keepdims=True), (8, 16))

    @pl.when(pl.program_id(0) == 0)
    def _():
        smax_ref[...] = blk

    @pl.when(pl.program_id(0) != 0)
    def _():
        smax_ref[...] = jnp.maximum(smax_ref[...], blk)


def _cfeat_body(ef_ref, wc_ref, bc_ref, ct_ref, cmax_ref):
    ct = jnp.dot(ef_ref[...], wc_ref[...],
                 preferred_element_type=jnp.float32)
    ct = ct + bc_ref[...]
    ct_ref[...] = ct
    blk = jnp.broadcast_to(jnp.max(ct, axis=0, keepdims=True), (8, 16))

    @pl.when(pl.program_id(0) == 0)
    def _():
        cmax_ref[...] = blk

    @pl.when(pl.program_id(0) != 0)
    def _():
        cmax_ref[...] = jnp.maximum(cmax_ref[...], blk)


def _ffn_body(num_ref, den_ref, p_ref, exp_ref, w1_ref, b1_ref, w2_ref,
              b2_ref, g_ref, b_ref, out_ref):
    den_exp = jnp.dot(den_ref[0] + den_ref[1], exp_ref[...],
                      preferred_element_type=jnp.float32)
    g = (num_ref[0] + num_ref[1]) / (den_exp + 1e-20)
    g = jnp.where(g > 0, g, jnp.exp(jnp.minimum(g, 0.0)) - 1.0)
    h = g + p_ref[...]
    mu = jnp.mean(h, axis=-1, keepdims=True)
    var = jnp.mean(jnp.square(h - mu), axis=-1, keepdims=True)
    xln = (h - mu) / jnp.sqrt(var + 1e-5) * g_ref[...] + b_ref[...]
    inter = jnp.dot(xln, w1_ref[...], preferred_element_type=jnp.float32)
    inter = jnp.maximum(inter + b1_ref[...], 0.0)
    out = jnp.dot(inter, w2_ref[...], preferred_element_type=jnp.float32)
    out_ref[...] = out + b2_ref[...] + h


def _dense_pre(w, edge_feat, fcW, attnW, featW, featB):
    a1 = attnW[:, :DH]
    a2 = attnW[:, DH:]
    fcw_cat = jnp.transpose(fcW, (1, 0, 2)).reshape(IN_DIM, H * DH)
    a1blk = jnp.zeros((H * DH, 16), jnp.float32)
    a1blk = a1blk.at[jnp.arange(H * DH), jnp.arange(H * DH) // DH].set(
        a1.reshape(-1))
    wc = jnp.einsum("hfd,hd->fh", featW, a2)
    bc = jnp.sum(featB * a2, axis=1)

    nb = N_W // BM
    zs, smax = pl.pallas_call(
        _zs_body,
        grid=(nb,),
        in_specs=[
            pl.BlockSpec((BM, IN_DIM), lambda i: (i, 0)),
            pl.BlockSpec((IN_DIM, H * DH), lambda i: (0, 0)),
            pl.BlockSpec((H * DH, 16), lambda i: (0, 0)),
        ],
        out_specs=[
            pl.BlockSpec((BM, 256), lambda i: (i, 0)),
            pl.BlockSpec((8, 16), lambda i: (0, 0)),
        ],
        out_shape=[
            jax.ShapeDtypeStruct((N_W, 256), jnp.float32),
            jax.ShapeDtypeStruct((8, 16), jnp.float32),
        ],
    )(w, fcw_cat, a1blk)

    ne = E // BE
    wc16 = jnp.concatenate([wc, jnp.zeros((FEAT, 8), jnp.float32)], axis=1)
    bc16 = jnp.concatenate([bc, jnp.zeros((8,), jnp.float32)])
    ct, cmax = pl.pallas_call(
        _cfeat_body,
        grid=(ne,),
        in_specs=[
            pl.BlockSpec((BE, FEAT), lambda i: (i, 0)),
            pl.BlockSpec((FEAT, 16), lambda i: (0, 0)),
            pl.BlockSpec((1, 16), lambda i: (0, 0)),
        ],
        out_specs=[
            pl.BlockSpec((BE, 16), lambda i: (i, 0)),
            pl.BlockSpec((8, 16), lambda i: (0, 0)),
        ],
        out_shape=[
            jax.ShapeDtypeStruct((E, 16), jnp.float32),
            jax.ShapeDtypeStruct((8, 16), jnp.float32),
        ],
    )(edge_feat, wc16, bc16.reshape(1, 16))

    s_max = smax[0, :H]
    c_max = cmax[0, :H]
    m = s_max + c_max
    m = jnp.where(m >= 0, m, 0.01 * m)
    m16 = jnp.concatenate([m, jnp.full((8,), 1e9, jnp.float32)])
    return zs, ct, m16


def _dense_post(num2, den2, p, w1W, w1b, w2W, w2b, ln_g, ln_b):
    expand = jnp.zeros((16, OUT_DIM), jnp.float32)
    expand = expand.at[jnp.arange(OUT_DIM) // DH, jnp.arange(OUT_DIM)].set(1.0)
    nb = N_P // BM
    return pl.pallas_call(
        _ffn_body,
        grid=(nb,),
        in_specs=[
            pl.BlockSpec((2, BM, OUT_DIM), lambda i: (0, i, 0)),
            pl.BlockSpec((2, BM, 16), lambda i: (0, i, 0)),
            pl.BlockSpec((BM, IN_DIM), lambda i: (i, 0)),
            pl.BlockSpec((16, OUT_DIM), lambda i: (0, 0)),
            pl.BlockSpec((OUT_DIM, FFN), lambda i: (0, 0)),
            pl.BlockSpec((1, FFN), lambda i: (0, 0)),
            pl.BlockSpec((FFN, OUT_DIM), lambda i: (0, 0)),
            pl.BlockSpec((1, OUT_DIM), lambda i: (0, 0)),
            pl.BlockSpec((1, OUT_DIM), lambda i: (0, 0)),
            pl.BlockSpec((1, OUT_DIM), lambda i: (0, 0)),
        ],
        out_specs=pl.BlockSpec((BM, OUT_DIM), lambda i: (i, 0)),
        out_shape=jax.ShapeDtypeStruct((N_P, OUT_DIM), jnp.float32),
    )(num2, den2, p, expand, w1W, w1b.reshape(1, FFN), w2W,
      w2b.reshape(1, OUT_DIM), ln_g.reshape(1, OUT_DIM),
      ln_b.reshape(1, OUT_DIM))


_K = 32
_NCHUNK = E // _K
_NTILE = 32
_MAXI = (_NCHUNK + _NTILE - 1) // _NTILE


def _vgather(vec, idx):
    dnums = lax.GatherDimensionNumbers(
        offset_dims=(), collapsed_slice_dims=(0,), start_index_map=(0,))
    return lax.gather(vec, idx[:, None], dnums, (1,),
                      mode=lax.GatherScatterMode.PROMISE_IN_BOUNDS)


def _edge_sc(zs, ct8, m128, src, dst):
    mesh = plsc.VectorSubcoreMesh(core_axis_name="c", subcore_axis_name="s")

    @functools.partial(
        pl.kernel,
        out_type=[jax.ShapeDtypeStruct((2 * N_P, 128), jnp.float32),
                  jax.ShapeDtypeStruct((2 * 1280, 128), jnp.float32)],
        mesh=mesh,
        scratch_types=[
            pltpu.VMEM((_K,), jnp.int32),
            pltpu.VMEM((_K,), jnp.int32),
            pltpu.VMEM((_K,), jnp.int32),
            pltpu.VMEM((_K + 16,), jnp.int32),
            pltpu.VMEM((_K, 256), jnp.float32),
            pltpu.VMEM((_K, 16), jnp.float32),
            pltpu.VMEM((_K, 128), jnp.float32),
            pltpu.VMEM((_K, 128), jnp.float32),
            pltpu.VMEM((128,), jnp.float32),
            pltpu.VMEM_SHARED((N_P, 128), jnp.float32),
            pltpu.VMEM_SHARED((1256, 128), jnp.float32),
            pltpu.SemaphoreType.DMA,
        ],
    )
    def k(zs_hbm, ct_hbm, m_hbm, src_hbm, dst_hbm, num_hbm, den_hbm,
          srcb, dstb, d8b, offb, rows, ctb, scl, dbuf, mv, num_sh, den_sh,
          sem):
        cid = lax.axis_index("c")
        sid = lax.axis_index("s")
        wid = cid * 16 + sid
        iota16 = lax.broadcasted_iota(jnp.int32, (16,), 0)
        zero16 = jnp.zeros((16,), jnp.float32)

        def _z_body(i, _):
            for j in range(8):
                scl[i, pl.ds(j * 16, 16)] = zero16
                dbuf[i, pl.ds(j * 16, 16)] = zero16
            return 0
        lax.fori_loop(0, _K, _z_body, 0, unroll=False)
        r0 = sid * 640
        d0 = sid * 80
        ro = cid * N_P + r0
        do = cid * 1280 + d0

        @pl.when(sid < 15)
        def _():
            for b in range(20):
                pltpu.sync_copy(scl, num_sh.at[pl.ds(r0 + b * _K, _K)])
            for b in range(2):
                pltpu.sync_copy(dbuf, den_sh.at[pl.ds(d0 + b * _K, _K)])
            pltpu.sync_copy(dbuf.at[pl.ds(0, 16)],
                            den_sh.at[pl.ds(d0 + 64, 16)])

        @pl.when(sid == 15)
        def _():
            for b in range(12):
                pltpu.sync_copy(scl, num_sh.at[pl.ds(r0 + b * _K, _K)])
            pltpu.sync_copy(scl.at[pl.ds(0, 16)],
                            num_sh.at[pl.ds(r0 + 384, 16)])
            pltpu.sync_copy(dbuf, den_sh.at[pl.ds(d0, _K)])
            pltpu.sync_copy(dbuf.at[pl.ds(0, 18)],
                            den_sh.at[pl.ds(d0 + 32, 18)])
        pltpu.sync_copy(m_hbm, mv)
        plsc.subcore_barrier()

        def _chunk(i, _):
            chunk = wid + i * _NTILE

            @pl.when(chunk < _NCHUNK)
            def _():
                cb = chunk * _K
                pltpu.sync_copy(src_hbm.at[pl.ds(cb, _K)], srcb)
                pltpu.sync_copy(dst_hbm.at[pl.ds(cb, _K)], dstb)
                pltpu.sync_copy(ct_hbm.at[pl.ds(cb, _K)], ctb)
                pltpu.async_copy(zs_hbm.at[srcb], rows, sem).wait()
                mvreg = mv[pl.ds(0, 16)]
                for g in range(_K // 16):
                    dv = dstb[pl.ds(g * 16, 16)]
                    d8b[pl.ds(g * 16, 16)] = lax.shift_right_logical(dv, 3)
                    offb[pl.ds(g * 16, 16)] = (dv & 7) * 16

                def _edge(kk, _):
                    srow = rows[kk, pl.ds(IN_DIM, 16)]
                    crow = ctb[kk, :]
                    x = srow + crow
                    e16 = jnp.maximum(x, 0.01 * x) - mvreg
                    exrow = jnp.exp(e16)
                    off = offb[pl.ds(kk, 16)][0]
                    dbuf[kk, pl.ds(off, 16)] = exrow
                    for h in range(8):
                        gkh = _vgather(exrow, iota16 * 0 + h)
                        zblk = rows[kk, pl.ds(h * 16, 16)]
                        scl[kk, pl.ds(h * 16, 16)] = zblk * gkh
                    return 0
                lax.fori_loop(0, _K, _edge, 0, unroll=False)
                pltpu.sync_copy(scl, num_sh.at[dstb], add=True)
                pltpu.sync_copy(dbuf, den_sh.at[d8b], add=True)

                def _wipe(kk, _):
                    off = offb[pl.ds(kk, 16)][0]
                    dbuf[kk, pl.ds(off, 16)] = zero16
                    return 0
                lax.fori_loop(0, _K, _wipe, 0, unroll=False)
            return 0
        lax.fori_loop(0, _MAXI, _chunk, 0, unroll=False)
        plsc.subcore_barrier()

        @pl.when(sid < 15)
        def _():
            pltpu.sync_copy(num_sh.at[pl.ds(r0, 640)],
                            num_hbm.at[pl.ds(ro, 640)])
            pltpu.sync_copy(den_sh.at[pl.ds(d0, 80)],
                            den_hbm.at[pl.ds(do, 80)])

        @pl.when(sid == 15)
        def _():
            pltpu.sync_copy(num_sh.at[pl.ds(r0, 400)],
                            num_hbm.at[pl.ds(ro, 400)])
            pltpu.sync_copy(den_sh.at[pl.ds(d0, 56)],
                            den_hbm.at[pl.ds(do, 56)])

    return k(zs, ct8, m128, src, dst)


def kernel(w, p, edge_feat, fcW, attnW, featW, featB, w1W, w1b, w2W, w2b,
           ln_g, ln_b, edge_index):
    src = edge_index[0].astype(jnp.int32)
    dst = edge_index[1].astype(jnp.int32)
    zs, ct8, m16 = _dense_pre(w, edge_feat, fcW, attnW, featW, featB)
    m128 = jnp.concatenate([m16, jnp.full((112,), 1e9, jnp.float32)])
    num, den = _edge_sc(zs, ct8, m128, src, dst)
    num2 = num.reshape(2, N_P, 128)
    den2 = den.reshape(2, 1280, 128)[:, :1250].reshape(2, N_P, 16)
    return _dense_post(num2, den2, p, w1W, w1b, w2W, w2b, ln_g, ln_b)
